# baseline (speedup 1.0000x reference)
import functools
import jax
import jax.numpy as jnp
from jax import lax
from jax.experimental import pallas as pl
from jax.experimental.pallas import tpu as pltpu
from jax.experimental.pallas import tpu_sc as plsc

B, N_FEAT, D = 16384, 100, 128
NP1 = N_FEAT + 1
NW = 32
ROWS_PER_W = B // NW
G = 2
XC = 16
NCHUNK = ROWS_PER_W // XC
GPC = XC // G
NV = D // 16

_mesh = plsc.VectorSubcoreMesh(core_axis_name="c", subcore_axis_name="s")


@functools.partial(
    pl.kernel,
    mesh=_mesh,
    out_type=jax.ShapeDtypeStruct((B, NP1, D), jnp.float32),
    scratch_types=[
        pltpu.VMEM((NP1, D), jnp.float32),
        pltpu.VMEM((N_FEAT, D), jnp.float32),
        pltpu.VMEM((XC, 128), jnp.float32),
        pltpu.VMEM((G, NP1, D), jnp.float32),
        pltpu.VMEM((G, NP1, D), jnp.float32),
        pltpu.SemaphoreType.DMA,
        pltpu.SemaphoreType.DMA,
    ],
)
def _sc_tok(x_hbm, w_hbm, b_hbm, out_hbm, w_v, b_v, x_v, o_v0, o_v1, sem0, sem1):
    c = lax.axis_index("c")
    s = lax.axis_index("s")
    wid = s * 2 + c
    base = wid * ROWS_PER_W
    pltpu.sync_copy(w_hbm, w_v)
    pltpu.sync_copy(b_hbm, b_v)
    o_bufs = (o_v0, o_v1)
    sems = (sem0, sem1)

    def compute_group(r_local, row0, o_v):
        def feat(n, carry2):
            for j in range(G):
                xs = x_v[r_local + j, pl.ds(n - 1, 16)][0]
                for dv in range(NV):
                    sl = pl.ds(dv * 16, 16)
                    o_v[j, n, sl] = xs * w_v[n, sl] + b_v[n - 1, sl]
            return carry2

        lax.fori_loop(1, NP1, feat, 0)
        for j in range(G):
            for dv in range(NV):
                sl = pl.ds(dv * 16, 16)
                o_v[j, 0, sl] = w_v[0, sl]

    def chunk(ci, carry):
        pltpu.sync_copy(x_hbm.at[pl.ds(base + ci * XC, XC)], x_v)

        def pair(q, carry2):
            for p in range(2):
                gl = q * 2 + p
                row0 = base + ci * XC + gl * G

                @pl.when((ci > 0) | (q > 0))
                def _wait():
                    pltpu.make_async_copy(
                        o_bufs[p], out_hbm.at[pl.ds(row0, G)], sems[p]
                    ).wait()

                compute_group(gl * G, row0, o_bufs[p])
                pltpu.make_async_copy(
                    o_bufs[p], out_hbm.at[pl.ds(row0, G)], sems[p]
                ).start()
            return carry2

        lax.fori_loop(0, GPC // 2, pair, 0)
        return carry

    lax.fori_loop(0, NCHUNK, chunk, 0)

    for p in range(2):
        row0 = base + ROWS_PER_W - (2 - p) * G
        pltpu.make_async_copy(
            o_bufs[p], out_hbm.at[pl.ds(row0, G)], sems[p]
        ).wait()


def kernel(x, numerical_weight, numerical_bias):
    x_pad = jnp.pad(x, ((0, 0), (0, 128 - N_FEAT)))
    return _sc_tok(x_pad, numerical_weight, numerical_bias)


# speedup vs baseline: 2.2695x; 2.2695x over previous
import jax
import jax.numpy as jnp
from jax.experimental import pallas as pl

B, N_FEAT, D_TOKEN = 16384, 100, 128
NP1 = N_FEAT + 1
BB = 64


def _tok_body(xn_ref, w_ref, b_ref, o_ref):
    xn = xn_ref[...]
    o_ref[...] = xn[:, :, None] * w_ref[...][None] + b_ref[...][None]


def kernel(x, numerical_weight, numerical_bias):
    ones = jnp.ones((x.shape[0], 1), dtype=x.dtype)
    xn = jnp.concatenate([ones, x], axis=1)
    zero = jnp.zeros((1, numerical_bias.shape[1]), dtype=numerical_bias.dtype)
    bias_p = jnp.concatenate([zero, numerical_bias], axis=0)

    return pl.pallas_call(
        _tok_body,
        grid=(B // BB,),
        in_specs=[
            pl.BlockSpec((BB, NP1), lambda i: (i, 0)),
            pl.BlockSpec((NP1, D_TOKEN), lambda i: (0, 0)),
            pl.BlockSpec((NP1, D_TOKEN), lambda i: (0, 0)),
        ],
        out_specs=pl.BlockSpec((BB, NP1, D_TOKEN), lambda i: (i, 0, 0)),
        out_shape=jax.ShapeDtypeStruct((B, NP1, D_TOKEN), x.dtype),
    )(xn, numerical_weight, bias_p)


# speedup vs baseline: 2.4675x; 1.0872x over previous
import jax
import jax.numpy as jnp
from jax.experimental import pallas as pl
from jax.experimental.pallas import tpu as pltpu

B, N_FEAT, D_TOKEN = 16384, 100, 128
NP1 = N_FEAT + 1
BB = 256


def _tok_body(xn_ref, w_ref, b_ref, o_ref):
    xn = xn_ref[...]
    o_ref[...] = xn[:, :, None] * w_ref[...][None] + b_ref[...][None]


def kernel(x, numerical_weight, numerical_bias):
    ones = jnp.ones((x.shape[0], 1), dtype=x.dtype)
    xn = jnp.concatenate([ones, x], axis=1)
    zero = jnp.zeros((1, numerical_bias.shape[1]), dtype=numerical_bias.dtype)
    bias_p = jnp.concatenate([zero, numerical_bias], axis=0)

    return pl.pallas_call(
        _tok_body,
        grid=(B // BB,),
        in_specs=[
            pl.BlockSpec((BB, NP1), lambda i: (i, 0)),
            pl.BlockSpec((NP1, D_TOKEN), lambda i: (0, 0)),
            pl.BlockSpec((NP1, D_TOKEN), lambda i: (0, 0)),
        ],
        out_specs=pl.BlockSpec((BB, NP1, D_TOKEN), lambda i: (i, 0, 0)),
        out_shape=jax.ShapeDtypeStruct((B, NP1, D_TOKEN), x.dtype),
        compiler_params=pltpu.CompilerParams(
            dimension_semantics=("parallel",),
        ),
    )(xn, numerical_weight, bias_p)
